# baseline (speedup 1.0000x reference)
import functools

import jax
import jax.numpy as jnp
from jax import lax
from jax.experimental import pallas as pl
from jax.experimental.pallas import tpu as pltpu
from jax.experimental.pallas import tpu_sc as plsc

T = 2048
D = 1024
H = 512
O = 1024
E = 64
K = 2
LOSS_COEF = 0.01

BT = 64
NTILES = (T * K) // BT + E
BCAP = NTILES * BT

NC = 2
NS = 16
NW = NC * NS

_CB = 256


def _gating_body(x_ref, wg_ref, dest_ref, gate_ref, eot_ref, loss_ref):
    logits = jnp.dot(x_ref[...], wg_ref[...], preferred_element_type=jnp.float32)
    col = lax.broadcasted_iota(jnp.int32, (T, E), 1)
    neg_inf = jnp.float32(-jnp.inf)

    m1 = jnp.max(logits, axis=1, keepdims=True)
    i1 = jnp.min(jnp.where(logits == m1, col, E), axis=1, keepdims=True)
    mask1 = col == i1
    l2 = jnp.where(mask1, neg_inf, logits)
    m2 = jnp.max(l2, axis=1, keepdims=True)
    i2 = jnp.min(jnp.where(l2 == m2, col, E), axis=1, keepdims=True)
    mask2 = col == i2

    e2 = jnp.exp(m2 - m1)
    denom = 1.0 + e2
    g1 = 1.0 / denom
    g2 = e2 / denom
    gate_ref[:, 0:1] = g1
    gate_ref[:, 1:2] = g2

    loadm = (mask1 | mask2).astype(jnp.float32)

    r = lax.broadcasted_iota(jnp.int32, (_CB, _CB), 0)
    c = lax.broadcasted_iota(jnp.int32, (_CB, _CB), 1)
    l_strict = (r > c).astype(jnp.float32)
    carry = jnp.zeros((1, E), jnp.float32)
    rk1, rk2 = [], []
    for b in range(T // _CB):
        blk = loadm[b * _CB:(b + 1) * _CB]
        excl = jnp.dot(l_strict, blk, preferred_element_type=jnp.float32) + carry
        m1b = mask1[b * _CB:(b + 1) * _CB].astype(jnp.float32)
        m2b = mask2[b * _CB:(b + 1) * _CB].astype(jnp.float32)
        rk1.append(jnp.sum(excl * m1b, axis=1, keepdims=True))
        rk2.append(jnp.sum(excl * m2b, axis=1, keepdims=True))
        carry = carry + jnp.sum(blk, axis=0, keepdims=True)
    rank1 = jnp.concatenate(rk1, axis=0)
    rank2 = jnp.concatenate(rk2, axis=0)
    load = carry

    ntiles_e = jnp.floor((load + (BT - 1)) * (1.0 / BT))
    er = lax.broadcasted_iota(jnp.int32, (E, E), 0)
    ec = lax.broadcasted_iota(jnp.int32, (E, E), 1)
    incl = (er <= ec).astype(jnp.float32)
    tile_cum = jnp.dot(ntiles_e, incl, preferred_element_type=jnp.float32)
    slot_off = (tile_cum - ntiles_e) * BT

    so1 = jnp.sum(mask1.astype(jnp.float32) * slot_off, axis=1, keepdims=True)
    so2 = jnp.sum(mask2.astype(jnp.float32) * slot_off, axis=1, keepdims=True)
    dest_ref[:, 0:1] = (so1 + rank1).astype(jnp.int32)
    dest_ref[:, 1:2] = (so2 + rank2).astype(jnp.int32)

    ti = lax.broadcasted_iota(jnp.int32, (NTILES, E), 0)
    tile_cum_i = jnp.broadcast_to(tile_cum.astype(jnp.int32), (NTILES, E))
    cmp = (tile_cum_i <= ti).astype(jnp.int32)
    eot_ref[...] = jnp.minimum(jnp.sum(cmp, axis=1, keepdims=True), E - 1)

    gmat = jnp.where(mask1, g1, 0.0) + jnp.where(mask2, g2, 0.0)
    importance = jnp.sum(gmat, axis=0, keepdims=True)

    def cv_sq(v):
        mean = jnp.sum(v) / E
        var = jnp.sum((v - mean) ** 2) / (E - 1)
        return var / (mean * mean + 1e-10)

    loss_val = (cv_sq(importance) + cv_sq(load)) * LOSS_COEF
    loss_ref[...] = jnp.broadcast_to(loss_val, (1, 1))


def _gating(x, w_gate):
    return pl.pallas_call(
        _gating_body,
        out_shape=[
            jax.ShapeDtypeStruct((T, K), jnp.int32),
            jax.ShapeDtypeStruct((T, K), jnp.float32),
            jax.ShapeDtypeStruct((NTILES, 1), jnp.int32),
            jax.ShapeDtypeStruct((1, 1), jnp.float32),
        ],
    )(x, w_gate)


_TOK_PER_W = T // NW


def _sc_dispatch(x, d0, d1):
    mesh = plsc.VectorSubcoreMesh(
        core_axis_name="c", subcore_axis_name="s", num_cores=NC, num_subcores=NS)

    @functools.partial(
        pl.kernel,
        mesh=mesh,
        out_type=jax.ShapeDtypeStruct((BCAP, D), jnp.float32),
        scratch_types=[
            pltpu.VMEM((_TOK_PER_W,), jnp.int32),
            pltpu.VMEM((_TOK_PER_W,), jnp.int32),
            pltpu.VMEM((_TOK_PER_W, D), jnp.float32),
            pltpu.SemaphoreType.DMA,
            pltpu.SemaphoreType.DMA,
        ],
    )
    def dispatch_kernel(x_hbm, d0_hbm, d1_hbm, out_hbm, i0_v, i1_v, rows_v,
                        s0, s1):
        wid = lax.axis_index("s") * NC + lax.axis_index("c")
        base = wid * _TOK_PER_W
        pltpu.sync_copy(d0_hbm.at[pl.ds(base, _TOK_PER_W)], i0_v)
        pltpu.sync_copy(d1_hbm.at[pl.ds(base, _TOK_PER_W)], i1_v)
        pltpu.sync_copy(x_hbm.at[pl.ds(base, _TOK_PER_W)], rows_v)
        c0 = pltpu.async_copy(rows_v, out_hbm.at[i0_v], s0)
        c1 = pltpu.async_copy(rows_v, out_hbm.at[i1_v], s1)
        c0.wait()
        c1.wait()

    return dispatch_kernel(x, d0, d1)


_DSPLIT = 2


def _mlp_body(eot_ref, xs_ref, w1_ref, b1_ref, w2_ref, b2_ref, ys_ref, acc_ref):
    del eot_ref
    s = pl.program_id(1)
    part = jnp.dot(xs_ref[...], w1_ref[0], preferred_element_type=jnp.float32)

    @pl.when(s == 0)
    def _first():
        acc_ref[...] = part

    @pl.when(s > 0)
    def _rest():
        acc_ref[...] += part

    @pl.when(s == _DSPLIT - 1)
    def _last():
        h = jnp.maximum(acc_ref[...] + b1_ref[0], 0.0)
        ys_ref[...] = jnp.dot(h, w2_ref[0], preferred_element_type=jnp.float32) + b2_ref[0]


def _mlp(expert_of_tile, xs, W1, b1, W2, b2):
    grid_spec = pltpu.PrefetchScalarGridSpec(
        num_scalar_prefetch=1,
        grid=(NTILES, _DSPLIT),
        in_specs=[
            pl.BlockSpec((BT, D // _DSPLIT), lambda i, s, eot: (i, s)),
            pl.BlockSpec((1, D // _DSPLIT, H), lambda i, s, eot: (eot[i], s, 0)),
            pl.BlockSpec((1, 1, H), lambda i, s, eot: (eot[i], 0, 0)),
            pl.BlockSpec((1, H, O), lambda i, s, eot: (eot[i], 0, 0)),
            pl.BlockSpec((1, 1, O), lambda i, s, eot: (eot[i], 0, 0)),
        ],
        out_specs=pl.BlockSpec((BT, O), lambda i, s, eot: (i, 0)),
        scratch_shapes=[pltpu.VMEM((BT, H), jnp.float32)],
    )
    return pl.pallas_call(
        _mlp_body,
        grid_spec=grid_spec,
        out_shape=jax.ShapeDtypeStruct((BCAP, O), jnp.float32),
        compiler_params=pltpu.CompilerParams(
            dimension_semantics=("arbitrary", "arbitrary")),
    )(expert_of_tile, xs, W1, b1.reshape(E, 1, H), W2, b2.reshape(E, 1, O))


def _sc_pair_gather(ys, p0, p1):
    mesh = plsc.VectorSubcoreMesh(
        core_axis_name="c", subcore_axis_name="s", num_cores=NC, num_subcores=NS)

    @functools.partial(
        pl.kernel,
        mesh=mesh,
        out_type=[
            jax.ShapeDtypeStruct((T, O), jnp.float32),
            jax.ShapeDtypeStruct((T, O), jnp.float32),
        ],
        scratch_types=[
            pltpu.VMEM((_TOK_PER_W,), jnp.int32),
            pltpu.VMEM((_TOK_PER_W,), jnp.int32),
            pltpu.VMEM((_TOK_PER_W, O), jnp.float32),
            pltpu.SemaphoreType.DMA,
        ],
    )
    def pair_gather_kernel(ys_hbm, p0_hbm, p1_hbm, a_hbm, b_hbm,
                           i0_v, i1_v, r_v, sem):
        wid = lax.axis_index("s") * NC + lax.axis_index("c")
        base = wid * _TOK_PER_W
        pltpu.sync_copy(p0_hbm.at[pl.ds(base, _TOK_PER_W)], i0_v)
        pltpu.sync_copy(p1_hbm.at[pl.ds(base, _TOK_PER_W)], i1_v)
        pltpu.async_copy(ys_hbm.at[i0_v], r_v, sem).wait()
        pltpu.sync_copy(r_v, a_hbm.at[pl.ds(base, _TOK_PER_W)])
        pltpu.async_copy(ys_hbm.at[i1_v], r_v, sem).wait()
        pltpu.sync_copy(r_v, b_hbm.at[pl.ds(base, _TOK_PER_W)])

    return pair_gather_kernel(ys, p0, p1)


_ABLK = 256


def _add_body(a_ref, b_ref, g_ref, o_ref):
    o_ref[...] = a_ref[...] * g_ref[:, 0:1] + b_ref[...] * g_ref[:, 1:2]


def _tc_combine(a, b, gates):
    return pl.pallas_call(
        _add_body,
        grid=(T // _ABLK,),
        in_specs=[
            pl.BlockSpec((_ABLK, O), lambda i: (i, 0)),
            pl.BlockSpec((_ABLK, O), lambda i: (i, 0)),
            pl.BlockSpec((_ABLK, K), lambda i: (i, 0)),
        ],
        out_specs=pl.BlockSpec((_ABLK, O), lambda i: (i, 0)),
        out_shape=jax.ShapeDtypeStruct((T, O), jnp.float32),
    )(a, b, gates)


def kernel(x, w_gate, W1, b1, W2, b2):
    dest, gates, eot, loss = _gating(x, w_gate)
    xs = _sc_dispatch(x, dest[:, 0], dest[:, 1])
    ys = _mlp(eot.reshape(-1), xs, W1, b1, W2, b2)
    a, b = _sc_pair_gather(ys, dest[:, 0], dest[:, 1])
    y = _tc_combine(a, b, gates)
    return (y, loss[0, 0])


# speedup vs baseline: 1.8542x; 1.8542x over previous
import functools

import jax
import jax.numpy as jnp
from jax import lax
from jax.experimental import pallas as pl
from jax.experimental.pallas import tpu as pltpu
from jax.experimental.pallas import tpu_sc as plsc

T = 2048
D = 1024
H = 512
O = 1024
E = 64
K = 2
LOSS_COEF = 0.01

BT = 128
NTILES = (T * K) // BT + E
BCAP = NTILES * BT

NC = 2
NS = 16
NW = NC * NS

_CB = 256


def _gating_body(x_ref, wg_ref, dest_ref, gate_ref, eot_ref, xsrc_ref, loss_ref):
    logits = jnp.dot(x_ref[...], wg_ref[...], preferred_element_type=jnp.float32)
    col = lax.broadcasted_iota(jnp.int32, (T, E), 1)
    neg_inf = jnp.float32(-jnp.inf)

    m1 = jnp.max(logits, axis=1, keepdims=True)
    i1 = jnp.min(jnp.where(logits == m1, col, E), axis=1, keepdims=True)
    mask1 = col == i1
    l2 = jnp.where(mask1, neg_inf, logits)
    m2 = jnp.max(l2, axis=1, keepdims=True)
    i2 = jnp.min(jnp.where(l2 == m2, col, E), axis=1, keepdims=True)
    mask2 = col == i2

    e2 = jnp.exp(m2 - m1)
    denom = 1.0 + e2
    g1 = 1.0 / denom
    g2 = e2 / denom
    gate_ref[:, 0:1] = g1
    gate_ref[:, 1:2] = g2

    loadm = (mask1 | mask2).astype(jnp.float32)

    r = lax.broadcasted_iota(jnp.int32, (_CB, _CB), 0)
    c = lax.broadcasted_iota(jnp.int32, (_CB, _CB), 1)
    l_strict = (r > c).astype(jnp.float32)
    carry = jnp.zeros((1, E), jnp.float32)
    rk1, rk2 = [], []
    for b in range(T // _CB):
        blk = loadm[b * _CB:(b + 1) * _CB]
        excl = jnp.dot(l_strict, blk, preferred_element_type=jnp.float32) + carry
        m1b = mask1[b * _CB:(b + 1) * _CB].astype(jnp.float32)
        m2b = mask2[b * _CB:(b + 1) * _CB].astype(jnp.float32)
        rk1.append(jnp.sum(excl * m1b, axis=1, keepdims=True))
        rk2.append(jnp.sum(excl * m2b, axis=1, keepdims=True))
        carry = carry + jnp.sum(blk, axis=0, keepdims=True)
    rank1 = jnp.concatenate(rk1, axis=0)
    rank2 = jnp.concatenate(rk2, axis=0)
    load = carry

    ntiles_e = jnp.floor((load + (BT - 1)) * (1.0 / BT))
    er = lax.broadcasted_iota(jnp.int32, (E, E), 0)
    ec = lax.broadcasted_iota(jnp.int32, (E, E), 1)
    incl = (er <= ec).astype(jnp.float32)
    tile_cum = jnp.dot(ntiles_e, incl, preferred_element_type=jnp.float32)
    slot_off = (tile_cum - ntiles_e) * BT

    so1 = jnp.sum(mask1.astype(jnp.float32) * slot_off, axis=1, keepdims=True)
    so2 = jnp.sum(mask2.astype(jnp.float32) * slot_off, axis=1, keepdims=True)
    dest_ref[:, 0:1] = (so1 + rank1).astype(jnp.int32)
    dest_ref[:, 1:2] = (so2 + rank2).astype(jnp.int32)

    n_used = tile_cum[0:1, E - 1:E].astype(jnp.int32)
    ti = lax.broadcasted_iota(jnp.int32, (NTILES, E), 0)
    src = jnp.minimum(ti, n_used[0, 0] - 1)
    xsrc_ref[...] = src[:, 0:1]
    tile_cum_i = jnp.broadcast_to(tile_cum.astype(jnp.int32), (NTILES, E))
    cmp = (tile_cum_i <= src).astype(jnp.int32)
    eot_ref[...] = jnp.minimum(jnp.sum(cmp, axis=1, keepdims=True), E - 1)

    gmat = jnp.where(mask1, g1, 0.0) + jnp.where(mask2, g2, 0.0)
    importance = jnp.sum(gmat, axis=0, keepdims=True)

    def cv_sq(v):
        mean = jnp.sum(v) / E
        var = jnp.sum((v - mean) ** 2) / (E - 1)
        return var / (mean * mean + 1e-10)

    loss_val = (cv_sq(importance) + cv_sq(load)) * LOSS_COEF
    loss_ref[...] = jnp.broadcast_to(loss_val, (1, 1))


def _gating(x, w_gate):
    return pl.pallas_call(
        _gating_body,
        out_shape=[
            jax.ShapeDtypeStruct((T, K), jnp.int32),
            jax.ShapeDtypeStruct((T, K), jnp.float32),
            jax.ShapeDtypeStruct((NTILES, 1), jnp.int32),
            jax.ShapeDtypeStruct((NTILES, 1), jnp.int32),
            jax.ShapeDtypeStruct((1, 1), jnp.float32),
        ],
    )(x, w_gate)


_TOK_PER_W = T // NW


def _sc_dispatch(x, d0, d1):
    mesh = plsc.VectorSubcoreMesh(
        core_axis_name="c", subcore_axis_name="s", num_cores=NC, num_subcores=NS)

    @functools.partial(
        pl.kernel,
        mesh=mesh,
        out_type=jax.ShapeDtypeStruct((BCAP, D), jnp.float32),
        scratch_types=[
            pltpu.VMEM((_TOK_PER_W,), jnp.int32),
            pltpu.VMEM((_TOK_PER_W,), jnp.int32),
            pltpu.VMEM((_TOK_PER_W, D), jnp.float32),
            pltpu.SemaphoreType.DMA,
            pltpu.SemaphoreType.DMA,
        ],
    )
    def dispatch_kernel(x_hbm, d0_hbm, d1_hbm, out_hbm, i0_v, i1_v, rows_v,
                        s0, s1):
        wid = lax.axis_index("s") * NC + lax.axis_index("c")
        base = wid * _TOK_PER_W
        pltpu.sync_copy(d0_hbm.at[pl.ds(base, _TOK_PER_W)], i0_v)
        pltpu.sync_copy(d1_hbm.at[pl.ds(base, _TOK_PER_W)], i1_v)
        pltpu.sync_copy(x_hbm.at[pl.ds(base, _TOK_PER_W)], rows_v)
        c0 = pltpu.async_copy(rows_v, out_hbm.at[i0_v], s0)
        c1 = pltpu.async_copy(rows_v, out_hbm.at[i1_v], s1)
        c0.wait()
        c1.wait()

    return dispatch_kernel(x, d0, d1)


def _mlp_body(eot_ref, xsrc_ref, xs_ref, w1_ref, b1_ref, w2_ref, b2_ref, ys_ref):
    del eot_ref
    i = pl.program_id(0)

    @pl.when(xsrc_ref[i] == i)
    def _active():
        h = jnp.dot(xs_ref[...], w1_ref[0], preferred_element_type=jnp.float32)
        h = jnp.maximum(h + b1_ref[0], 0.0)
        ys_ref[...] = jnp.dot(h, w2_ref[0], preferred_element_type=jnp.float32) + b2_ref[0]


def _mlp(expert_of_tile, xsrc, xs, W1, b1, W2, b2):
    grid_spec = pltpu.PrefetchScalarGridSpec(
        num_scalar_prefetch=2,
        grid=(NTILES,),
        in_specs=[
            pl.BlockSpec((BT, D), lambda i, eot, xsrc: (xsrc[i], 0)),
            pl.BlockSpec((1, D, H), lambda i, eot, xsrc: (eot[i], 0, 0)),
            pl.BlockSpec((1, 1, H), lambda i, eot, xsrc: (eot[i], 0, 0)),
            pl.BlockSpec((1, H, O), lambda i, eot, xsrc: (eot[i], 0, 0)),
            pl.BlockSpec((1, 1, O), lambda i, eot, xsrc: (eot[i], 0, 0)),
        ],
        out_specs=pl.BlockSpec((BT, O), lambda i, eot, xsrc: (xsrc[i], 0)),
    )
    return pl.pallas_call(
        _mlp_body,
        grid_spec=grid_spec,
        out_shape=jax.ShapeDtypeStruct((BCAP, O), jnp.float32),
        compiler_params=pltpu.CompilerParams(
            dimension_semantics=("arbitrary",)),
    )(expert_of_tile, xsrc, xs, W1, b1.reshape(E, 1, H), W2, b2.reshape(E, 1, O))


def _sc_pair_gather(ys, p0, p1):
    mesh = plsc.VectorSubcoreMesh(
        core_axis_name="c", subcore_axis_name="s", num_cores=NC, num_subcores=NS)

    @functools.partial(
        pl.kernel,
        mesh=mesh,
        out_type=[
            jax.ShapeDtypeStruct((T, O), jnp.float32),
            jax.ShapeDtypeStruct((T, O), jnp.float32),
        ],
        scratch_types=[
            pltpu.VMEM((_TOK_PER_W,), jnp.int32),
            pltpu.VMEM((_TOK_PER_W,), jnp.int32),
            pltpu.VMEM((_TOK_PER_W, O), jnp.float32),
            pltpu.SemaphoreType.DMA,
        ],
    )
    def pair_gather_kernel(ys_hbm, p0_hbm, p1_hbm, a_hbm, b_hbm,
                           i0_v, i1_v, r_v, sem):
        wid = lax.axis_index("s") * NC + lax.axis_index("c")
        base = wid * _TOK_PER_W
        pltpu.sync_copy(p0_hbm.at[pl.ds(base, _TOK_PER_W)], i0_v)
        pltpu.sync_copy(p1_hbm.at[pl.ds(base, _TOK_PER_W)], i1_v)
        pltpu.async_copy(ys_hbm.at[i0_v], r_v, sem).wait()
        pltpu.sync_copy(r_v, a_hbm.at[pl.ds(base, _TOK_PER_W)])
        pltpu.async_copy(ys_hbm.at[i1_v], r_v, sem).wait()
        pltpu.sync_copy(r_v, b_hbm.at[pl.ds(base, _TOK_PER_W)])

    return pair_gather_kernel(ys, p0, p1)


_ABLK = 256


def _add_body(a_ref, b_ref, g_ref, o_ref):
    o_ref[...] = a_ref[...] * g_ref[:, 0:1] + b_ref[...] * g_ref[:, 1:2]


def _tc_combine(a, b, gates):
    return pl.pallas_call(
        _add_body,
        grid=(T // _ABLK,),
        in_specs=[
            pl.BlockSpec((_ABLK, O), lambda i: (i, 0)),
            pl.BlockSpec((_ABLK, O), lambda i: (i, 0)),
            pl.BlockSpec((_ABLK, K), lambda i: (i, 0)),
        ],
        out_specs=pl.BlockSpec((_ABLK, O), lambda i: (i, 0)),
        out_shape=jax.ShapeDtypeStruct((T, O), jnp.float32),
    )(a, b, gates)


def kernel(x, w_gate, W1, b1, W2, b2):
    dest, gates, eot, xsrc, loss = _gating(x, w_gate)
    xs = _sc_dispatch(x, dest[:, 0], dest[:, 1])
    ys = _mlp(eot.reshape(-1), xsrc.reshape(-1), xs, W1, b1, W2, b2)
    a, b = _sc_pair_gather(ys, dest[:, 0], dest[:, 1])
    y = _tc_combine(a, b, gates)
    return (y, loss[0, 0])
